# baseline (speedup 1.0000x reference)
import functools

import jax
import jax.numpy as jnp
from jax import lax
from jax.experimental import pallas as pl
from jax.experimental.pallas import tpu as pltpu
from jax.experimental.pallas import tpu_sc as plsc

_B, _C, _HW = 1024, 10, 4096
_P = 128
_PSC = 1024
_PT = _HW // 8
_BT = _B // 128
_PTSC = _PSC // 8
_NW = 32
_PWT = _PTSC // _NW
_TROW = _BT * 8 * 128
_CCH = 2
_NV = _C // _CCH
_NT = _PWT * _NV
_GCH = _CCH * _TROW
_L = 16


def _fill_body(g_ref, m_ref, par_ref, o_ref):
    pred = m_ref[...] > par_ref[_C]
    for c in range(_C):
        o_ref[c] = jnp.where(pred, par_ref[c], g_ref[c])


def _tc_fill(gT, mT, params):
    base = _PSC // _P
    return pl.pallas_call(
        _fill_body,
        grid=((_HW - _PSC) // _P,),
        in_specs=[
            pl.BlockSpec((_C, _P, _B), lambda j: (0, j + base, 0)),
            pl.BlockSpec((_P, _B), lambda j: (j + base, 0)),
            pl.BlockSpec(memory_space=pltpu.SMEM),
        ],
        out_specs=pl.BlockSpec((_C, _P, _B), lambda j: (0, j + base, 0)),
        out_shape=jax.ShapeDtypeStruct((_C, _HW, _B), jnp.float32),
        compiler_params=pltpu.CompilerParams(
            dimension_semantics=("arbitrary",)),
    )(gT, mT, params)


def _build_sc_call():
    mesh = plsc.VectorSubcoreMesh(core_axis_name="c", subcore_axis_name="s")

    @functools.partial(
        pl.kernel,
        out_type=jax.ShapeDtypeStruct((_C, _PTSC, _BT, 8, 128), jnp.float32),
        mesh=mesh,
        scratch_types=[
            pltpu.VMEM((_CCH, 1, _BT, 8, 128), jnp.float32),
            pltpu.VMEM((_CCH, 1, _BT, 8, 128), jnp.float32),
            pltpu.VMEM((_CCH, 1, _BT, 8, 128), jnp.float32),
            pltpu.VMEM((_CCH, 1, _BT, 8, 128), jnp.float32),
            pltpu.VMEM((1, _BT, 8, 128), jnp.float32),
            pltpu.VMEM((1, _BT, 8, 128), jnp.float32),
            pltpu.VMEM((_C * _L,), jnp.float32),
            pltpu.VMEM((_L,), jnp.float32),
            pltpu.SemaphoreType.DMA,
            pltpu.SemaphoreType.DMA,
            pltpu.SemaphoreType.DMA,
            pltpu.SemaphoreType.DMA,
            pltpu.SemaphoreType.DMA,
            pltpu.SemaphoreType.DMA,
        ],
    )
    def sc_fill(g5, m4, fill_h, thr_h, out5,
                gin0, gin1, gout0, gout1, m0, m1, fvm, tvm,
                sg0, sg1, sm0, sm1, so0, so1):
        wid = lax.axis_index("s") * 2 + lax.axis_index("c")
        pt0 = wid * _PWT

        pltpu.sync_copy(fill_h, fvm)
        pltpu.sync_copy(thr_h, tvm)
        thrv = tvm[...]
        fills = [fvm[pl.ds(c * _L, _L)] for c in range(_C)]

        gins, gouts, ms = (gin0, gin1), (gout0, gout1), (m0, m1)
        sgs, sms, sos = (sg0, sg1), (sm0, sm1), (so0, so1)

        def gin_desc(t, s):
            u, v = divmod(t, _NV)
            return pltpu.make_async_copy(
                g5.at[pl.ds(_CCH * v, _CCH), pl.ds(pt0 + u, 1)],
                gins[s], sgs[s])

        def gout_desc(t, s):
            u, v = divmod(t, _NV)
            return pltpu.make_async_copy(
                gouts[s],
                out5.at[pl.ds(_CCH * v, _CCH), pl.ds(pt0 + u, 1)], sos[s])

        def m_desc(u):
            return pltpu.make_async_copy(
                m4.at[pl.ds(pt0 + u, 1)], ms[u % 2], sms[u % 2])

        def compute(t, s):
            u, v = divmod(t, _NV)
            gi = gins[s].reshape(_CCH, _TROW)
            go = gouts[s].reshape(_CCH, _TROW)
            mb = ms[u % 2].reshape(1, _TROW)
            cfills = fills[_CCH * v:_CCH * (v + 1)]

            def jbody(j, carry):
                o = j * _L
                pred = mb[0, pl.ds(o, _L)] > thrv
                for ci in range(_CCH):
                    go[ci, pl.ds(o, _L)] = jnp.where(
                        pred, cfills[ci], gi[ci, pl.ds(o, _L)])
                return carry

            lax.fori_loop(0, _TROW // _L, jbody, 0, unroll=2)

        m_desc(0).start()
        gin_desc(0, 0).start()
        gin_desc(1, 1).start()
        for t in range(_NT):
            s = t % 2
            u, v = divmod(t, _NV)
            if v == 0:
                m_desc(u).wait()
                if u + 1 < _PWT:
                    m_desc(u + 1).start()
            gin_desc(t, s).wait()
            if t >= 2:
                gout_desc(t - 2, s).wait()
            compute(t, s)
            gout_desc(t, s).start()
            if t + 2 < _NT:
                gin_desc(t + 2, s).start()
        gout_desc(_NT - 2, 0).wait()
        gout_desc(_NT - 1, 1).wait()

    return sc_fill


_sc_fill = _build_sc_call()


def kernel(grid, mask, color):
    gT = jnp.transpose(grid, (1, 2, 3, 0)).reshape(_C, _HW, _B)
    mT = jnp.transpose(mask, (1, 2, 3, 0)).reshape(_HW, _B)
    g5 = gT.reshape(_C, _PT, 8, _BT, 128).transpose(0, 1, 3, 2, 4)
    m4 = mT.reshape(_PT, 8, _BT, 128).transpose(0, 2, 1, 3)

    color = jnp.asarray(color)
    valid = (color >= 0) & (color < _C)
    safe = jnp.clip(color, 0, _C - 1)
    fill = (jnp.arange(_C) == safe).astype(jnp.float32)
    thr = jnp.where(valid, jnp.float32(0.5), jnp.float32(3.0e38))
    params = jnp.concatenate([fill, thr[None]])
    fill16 = jnp.broadcast_to(fill[:, None], (_C, _L)).reshape(_C * _L)
    thr16 = jnp.broadcast_to(thr, (_L,))

    sc5 = _sc_fill(g5, m4, fill16, thr16)
    sc3 = sc5.transpose(0, 1, 3, 2, 4).reshape(_C, _PSC, _B)
    tc_full = _tc_fill(gT, mT, params)
    out3 = lax.dynamic_update_slice(tc_full, sc3, (0, 0, 0))
    return jnp.transpose(out3.reshape(_C, 64, 64, _B), (3, 0, 1, 2))


# speedup vs baseline: 1.1037x; 1.1037x over previous
import functools

import jax
import jax.numpy as jnp
from jax import lax
from jax.experimental import pallas as pl
from jax.experimental.pallas import tpu as pltpu
from jax.experimental.pallas import tpu_sc as plsc

_B, _C, _HW = 1024, 10, 4096
_P = 128
_PSC = 512
_PT = _HW // 8
_BT = _B // 128
_PTSC = _PSC // 8
_NW = 32
_PWT = _PTSC // _NW
_TROW = _BT * 8 * 128
_CCH = 2
_NV = _C // _CCH
_NT = _PWT * _NV
_GCH = _CCH * _TROW
_L = 16


def _fill_body(g_ref, m_ref, par_ref, o_ref):
    pred = m_ref[...] > par_ref[_C]
    for c in range(_C):
        o_ref[c] = jnp.where(pred, par_ref[c], g_ref[c])


def _tc_fill(gT, mT, params):
    base = _PSC // _P
    return pl.pallas_call(
        _fill_body,
        grid=((_HW - _PSC) // _P,),
        in_specs=[
            pl.BlockSpec((_C, _P, _B), lambda j: (0, j + base, 0)),
            pl.BlockSpec((_P, _B), lambda j: (j + base, 0)),
            pl.BlockSpec(memory_space=pltpu.SMEM),
        ],
        out_specs=pl.BlockSpec((_C, _P, _B), lambda j: (0, j + base, 0)),
        out_shape=jax.ShapeDtypeStruct((_C, _HW, _B), jnp.float32),
        compiler_params=pltpu.CompilerParams(
            dimension_semantics=("arbitrary",)),
    )(gT, mT, params)


def _build_sc_call():
    mesh = plsc.VectorSubcoreMesh(core_axis_name="c", subcore_axis_name="s")

    @functools.partial(
        pl.kernel,
        out_type=jax.ShapeDtypeStruct((_C, _PTSC, _BT, 8, 128), jnp.float32),
        mesh=mesh,
        scratch_types=[
            pltpu.VMEM((_CCH, 1, _BT, 8, 128), jnp.float32),
            pltpu.VMEM((_CCH, 1, _BT, 8, 128), jnp.float32),
            pltpu.VMEM((_CCH, 1, _BT, 8, 128), jnp.float32),
            pltpu.VMEM((_CCH, 1, _BT, 8, 128), jnp.float32),
            pltpu.VMEM((1, _BT, 8, 128), jnp.float32),
            pltpu.VMEM((1, _BT, 8, 128), jnp.float32),
            pltpu.VMEM((_C * _L,), jnp.float32),
            pltpu.VMEM((_L,), jnp.float32),
            pltpu.SemaphoreType.DMA,
            pltpu.SemaphoreType.DMA,
            pltpu.SemaphoreType.DMA,
            pltpu.SemaphoreType.DMA,
            pltpu.SemaphoreType.DMA,
            pltpu.SemaphoreType.DMA,
        ],
    )
    def sc_fill(g5, m4, fill_h, thr_h, out5,
                gin0, gin1, gout0, gout1, m0, m1, fvm, tvm,
                sg0, sg1, sm0, sm1, so0, so1):
        wid = lax.axis_index("s") * 2 + lax.axis_index("c")
        pt0 = wid * _PWT

        pltpu.sync_copy(fill_h, fvm)
        pltpu.sync_copy(thr_h, tvm)
        thrv = tvm[...]
        fills = [fvm[pl.ds(c * _L, _L)] for c in range(_C)]

        gins, gouts, ms = (gin0, gin1), (gout0, gout1), (m0, m1)
        sgs, sms, sos = (sg0, sg1), (sm0, sm1), (so0, so1)

        def gin_desc(t, s):
            u, v = divmod(t, _NV)
            return pltpu.make_async_copy(
                g5.at[pl.ds(_CCH * v, _CCH), pl.ds(pt0 + u, 1)],
                gins[s], sgs[s])

        def gout_desc(t, s):
            u, v = divmod(t, _NV)
            return pltpu.make_async_copy(
                gouts[s],
                out5.at[pl.ds(_CCH * v, _CCH), pl.ds(pt0 + u, 1)], sos[s])

        def m_desc(u):
            return pltpu.make_async_copy(
                m4.at[pl.ds(pt0 + u, 1)], ms[u % 2], sms[u % 2])

        def compute(t, s):
            u, v = divmod(t, _NV)
            gi = gins[s].reshape(_CCH, _TROW)
            go = gouts[s].reshape(_CCH, _TROW)
            mb = ms[u % 2].reshape(1, _TROW)
            cfills = fills[_CCH * v:_CCH * (v + 1)]

            def jbody(j, carry):
                o = j * _L
                pred = mb[0, pl.ds(o, _L)] > thrv
                for ci in range(_CCH):
                    go[ci, pl.ds(o, _L)] = jnp.where(
                        pred, cfills[ci], gi[ci, pl.ds(o, _L)])
                return carry

            lax.fori_loop(0, _TROW // _L, jbody, 0, unroll=2)

        m_desc(0).start()
        gin_desc(0, 0).start()
        gin_desc(1, 1).start()
        for t in range(_NT):
            s = t % 2
            u, v = divmod(t, _NV)
            if v == 0:
                m_desc(u).wait()
                if u + 1 < _PWT:
                    m_desc(u + 1).start()
            gin_desc(t, s).wait()
            if t >= 2:
                gout_desc(t - 2, s).wait()
            compute(t, s)
            gout_desc(t, s).start()
            if t + 2 < _NT:
                gin_desc(t + 2, s).start()
        gout_desc(_NT - 2, 0).wait()
        gout_desc(_NT - 1, 1).wait()

    return sc_fill


_sc_fill = _build_sc_call()


def kernel(grid, mask, color):
    gT = jnp.transpose(grid, (1, 2, 3, 0)).reshape(_C, _HW, _B)
    mT = jnp.transpose(mask, (1, 2, 3, 0)).reshape(_HW, _B)
    g5 = gT.reshape(_C, _PT, 8, _BT, 128).transpose(0, 1, 3, 2, 4)
    m4 = mT.reshape(_PT, 8, _BT, 128).transpose(0, 2, 1, 3)

    color = jnp.asarray(color)
    valid = (color >= 0) & (color < _C)
    safe = jnp.clip(color, 0, _C - 1)
    fill = (jnp.arange(_C) == safe).astype(jnp.float32)
    thr = jnp.where(valid, jnp.float32(0.5), jnp.float32(3.0e38))
    params = jnp.concatenate([fill, thr[None]])
    fill16 = jnp.broadcast_to(fill[:, None], (_C, _L)).reshape(_C * _L)
    thr16 = jnp.broadcast_to(thr, (_L,))

    sc5 = _sc_fill(g5, m4, fill16, thr16)
    sc3 = sc5.transpose(0, 1, 3, 2, 4).reshape(_C, _PSC, _B)
    tc_full = _tc_fill(gT, mT, params)
    out3 = lax.dynamic_update_slice(tc_full, sc3, (0, 0, 0))
    return jnp.transpose(out3.reshape(_C, 64, 64, _B), (3, 0, 1, 2))


# speedup vs baseline: 1.1464x; 1.0387x over previous
import functools

import jax
import jax.numpy as jnp
from jax import lax
from jax.experimental import pallas as pl
from jax.experimental.pallas import tpu as pltpu
from jax.experimental.pallas import tpu_sc as plsc

_B, _C, _HW = 1024, 10, 4096
_P = 128
_PSC = 256
_PT = _HW // 8
_BT = _B // 128
_PTSC = _PSC // 8
_NW = 32
_PWT = _PTSC // _NW
_TROW = _BT * 8 * 128
_CCH = 2
_NV = _C // _CCH
_NT = _PWT * _NV
_GCH = _CCH * _TROW
_L = 16


def _fill_body(g_ref, m_ref, par_ref, o_ref):
    pred = m_ref[...] > par_ref[_C]
    for c in range(_C):
        o_ref[c] = jnp.where(pred, par_ref[c], g_ref[c])


def _tc_fill(gT, mT, params):
    base = _PSC // _P
    return pl.pallas_call(
        _fill_body,
        grid=((_HW - _PSC) // _P,),
        in_specs=[
            pl.BlockSpec((_C, _P, _B), lambda j: (0, j + base, 0)),
            pl.BlockSpec((_P, _B), lambda j: (j + base, 0)),
            pl.BlockSpec(memory_space=pltpu.SMEM),
        ],
        out_specs=pl.BlockSpec((_C, _P, _B), lambda j: (0, j + base, 0)),
        out_shape=jax.ShapeDtypeStruct((_C, _HW, _B), jnp.float32),
        compiler_params=pltpu.CompilerParams(
            dimension_semantics=("arbitrary",)),
    )(gT, mT, params)


def _build_sc_call():
    mesh = plsc.VectorSubcoreMesh(core_axis_name="c", subcore_axis_name="s")

    @functools.partial(
        pl.kernel,
        out_type=jax.ShapeDtypeStruct((_C, _PTSC, _BT, 8, 128), jnp.float32),
        mesh=mesh,
        scratch_types=[
            pltpu.VMEM((_CCH, 1, _BT, 8, 128), jnp.float32),
            pltpu.VMEM((_CCH, 1, _BT, 8, 128), jnp.float32),
            pltpu.VMEM((_CCH, 1, _BT, 8, 128), jnp.float32),
            pltpu.VMEM((_CCH, 1, _BT, 8, 128), jnp.float32),
            pltpu.VMEM((1, _BT, 8, 128), jnp.float32),
            pltpu.VMEM((1, _BT, 8, 128), jnp.float32),
            pltpu.VMEM((_C * _L,), jnp.float32),
            pltpu.VMEM((_L,), jnp.float32),
            pltpu.SemaphoreType.DMA,
            pltpu.SemaphoreType.DMA,
            pltpu.SemaphoreType.DMA,
            pltpu.SemaphoreType.DMA,
            pltpu.SemaphoreType.DMA,
            pltpu.SemaphoreType.DMA,
        ],
    )
    def sc_fill(g5, m4, fill_h, thr_h, out5,
                gin0, gin1, gout0, gout1, m0, m1, fvm, tvm,
                sg0, sg1, sm0, sm1, so0, so1):
        wid = lax.axis_index("s") * 2 + lax.axis_index("c")
        pt0 = wid * _PWT

        pltpu.sync_copy(fill_h, fvm)
        pltpu.sync_copy(thr_h, tvm)
        thrv = tvm[...]
        fills = [fvm[pl.ds(c * _L, _L)] for c in range(_C)]

        gins, gouts, ms = (gin0, gin1), (gout0, gout1), (m0, m1)
        sgs, sms, sos = (sg0, sg1), (sm0, sm1), (so0, so1)

        def gin_desc(t, s):
            u, v = divmod(t, _NV)
            return pltpu.make_async_copy(
                g5.at[pl.ds(_CCH * v, _CCH), pl.ds(pt0 + u, 1)],
                gins[s], sgs[s])

        def gout_desc(t, s):
            u, v = divmod(t, _NV)
            return pltpu.make_async_copy(
                gouts[s],
                out5.at[pl.ds(_CCH * v, _CCH), pl.ds(pt0 + u, 1)], sos[s])

        def m_desc(u):
            return pltpu.make_async_copy(
                m4.at[pl.ds(pt0 + u, 1)], ms[u % 2], sms[u % 2])

        def compute(t, s):
            u, v = divmod(t, _NV)
            gi = gins[s].reshape(_CCH, _TROW)
            go = gouts[s].reshape(_CCH, _TROW)
            mb = ms[u % 2].reshape(1, _TROW)
            cfills = fills[_CCH * v:_CCH * (v + 1)]

            def jbody(j, carry):
                o = j * _L
                pred = mb[0, pl.ds(o, _L)] > thrv
                for ci in range(_CCH):
                    go[ci, pl.ds(o, _L)] = jnp.where(
                        pred, cfills[ci], gi[ci, pl.ds(o, _L)])
                return carry

            lax.fori_loop(0, _TROW // _L, jbody, 0, unroll=2)

        m_desc(0).start()
        gin_desc(0, 0).start()
        gin_desc(1, 1).start()
        for t in range(_NT):
            s = t % 2
            u, v = divmod(t, _NV)
            if v == 0:
                m_desc(u).wait()
                if u + 1 < _PWT:
                    m_desc(u + 1).start()
            gin_desc(t, s).wait()
            if t >= 2:
                gout_desc(t - 2, s).wait()
            compute(t, s)
            gout_desc(t, s).start()
            if t + 2 < _NT:
                gin_desc(t + 2, s).start()
        gout_desc(_NT - 2, (_NT - 2) % 2).wait()
        gout_desc(_NT - 1, (_NT - 1) % 2).wait()

    return sc_fill


_sc_fill = _build_sc_call()


def kernel(grid, mask, color):
    gT = jnp.transpose(grid, (1, 2, 3, 0)).reshape(_C, _HW, _B)
    mT = jnp.transpose(mask, (1, 2, 3, 0)).reshape(_HW, _B)
    g5 = gT.reshape(_C, _PT, 8, _BT, 128).transpose(0, 1, 3, 2, 4)
    m4 = mT.reshape(_PT, 8, _BT, 128).transpose(0, 2, 1, 3)

    color = jnp.asarray(color)
    valid = (color >= 0) & (color < _C)
    safe = jnp.clip(color, 0, _C - 1)
    fill = (jnp.arange(_C) == safe).astype(jnp.float32)
    thr = jnp.where(valid, jnp.float32(0.5), jnp.float32(3.0e38))
    params = jnp.concatenate([fill, thr[None]])
    fill16 = jnp.broadcast_to(fill[:, None], (_C, _L)).reshape(_C * _L)
    thr16 = jnp.broadcast_to(thr, (_L,))

    sc5 = _sc_fill(g5, m4, fill16, thr16)
    sc3 = sc5.transpose(0, 1, 3, 2, 4).reshape(_C, _PSC, _B)
    tc_full = _tc_fill(gT, mT, params)
    out3 = lax.dynamic_update_slice(tc_full, sc3, (0, 0, 0))
    return jnp.transpose(out3.reshape(_C, 64, 64, _B), (3, 0, 1, 2))


# speedup vs baseline: 1.1528x; 1.0056x over previous
import functools

import jax
import jax.numpy as jnp
from jax import lax
from jax.experimental import pallas as pl
from jax.experimental.pallas import tpu as pltpu
from jax.experimental.pallas import tpu_sc as plsc

_B, _C, _HW = 1024, 10, 4096
_P = 256
_PSC = 256
_PT = _HW // 8
_BT = _B // 128
_PTSC = _PSC // 8
_NW = 32
_PWT = _PTSC // _NW
_TROW = _BT * 8 * 128
_CCH = 2
_NV = _C // _CCH
_NT = _PWT * _NV
_GCH = _CCH * _TROW
_L = 16


def _fill_body(g_ref, m_ref, par_ref, o_ref):
    pred = m_ref[...] > par_ref[_C]
    for c in range(_C):
        o_ref[c] = jnp.where(pred, par_ref[c], g_ref[c])


def _tc_fill(gT, mT, params):
    base = _PSC // _P
    return pl.pallas_call(
        _fill_body,
        grid=((_HW - _PSC) // _P,),
        in_specs=[
            pl.BlockSpec((_C, _P, _B), lambda j: (0, j + base, 0)),
            pl.BlockSpec((_P, _B), lambda j: (j + base, 0)),
            pl.BlockSpec(memory_space=pltpu.SMEM),
        ],
        out_specs=pl.BlockSpec((_C, _P, _B), lambda j: (0, j + base, 0)),
        out_shape=jax.ShapeDtypeStruct((_C, _HW, _B), jnp.float32),
        compiler_params=pltpu.CompilerParams(
            dimension_semantics=("arbitrary",)),
    )(gT, mT, params)


def _build_sc_call():
    mesh = plsc.VectorSubcoreMesh(core_axis_name="c", subcore_axis_name="s")

    @functools.partial(
        pl.kernel,
        out_type=jax.ShapeDtypeStruct((_C, _PTSC, _BT, 8, 128), jnp.float32),
        mesh=mesh,
        scratch_types=[
            pltpu.VMEM((_CCH, 1, _BT, 8, 128), jnp.float32),
            pltpu.VMEM((_CCH, 1, _BT, 8, 128), jnp.float32),
            pltpu.VMEM((_CCH, 1, _BT, 8, 128), jnp.float32),
            pltpu.VMEM((_CCH, 1, _BT, 8, 128), jnp.float32),
            pltpu.VMEM((1, _BT, 8, 128), jnp.float32),
            pltpu.VMEM((1, _BT, 8, 128), jnp.float32),
            pltpu.VMEM((_C * _L,), jnp.float32),
            pltpu.VMEM((_L,), jnp.float32),
            pltpu.SemaphoreType.DMA,
            pltpu.SemaphoreType.DMA,
            pltpu.SemaphoreType.DMA,
            pltpu.SemaphoreType.DMA,
            pltpu.SemaphoreType.DMA,
            pltpu.SemaphoreType.DMA,
        ],
    )
    def sc_fill(g5, m4, fill_h, thr_h, out5,
                gin0, gin1, gout0, gout1, m0, m1, fvm, tvm,
                sg0, sg1, sm0, sm1, so0, so1):
        wid = lax.axis_index("s") * 2 + lax.axis_index("c")
        pt0 = wid * _PWT

        pltpu.sync_copy(fill_h, fvm)
        pltpu.sync_copy(thr_h, tvm)
        thrv = tvm[...]
        fills = [fvm[pl.ds(c * _L, _L)] for c in range(_C)]

        gins, gouts, ms = (gin0, gin1), (gout0, gout1), (m0, m1)
        sgs, sms, sos = (sg0, sg1), (sm0, sm1), (so0, so1)

        def gin_desc(t, s):
            u, v = divmod(t, _NV)
            return pltpu.make_async_copy(
                g5.at[pl.ds(_CCH * v, _CCH), pl.ds(pt0 + u, 1)],
                gins[s], sgs[s])

        def gout_desc(t, s):
            u, v = divmod(t, _NV)
            return pltpu.make_async_copy(
                gouts[s],
                out5.at[pl.ds(_CCH * v, _CCH), pl.ds(pt0 + u, 1)], sos[s])

        def m_desc(u):
            return pltpu.make_async_copy(
                m4.at[pl.ds(pt0 + u, 1)], ms[u % 2], sms[u % 2])

        def compute(t, s):
            u, v = divmod(t, _NV)
            gi = gins[s].reshape(_CCH, _TROW)
            go = gouts[s].reshape(_CCH, _TROW)
            mb = ms[u % 2].reshape(1, _TROW)
            cfills = fills[_CCH * v:_CCH * (v + 1)]

            def jbody(j, carry):
                o = j * _L
                pred = mb[0, pl.ds(o, _L)] > thrv
                for ci in range(_CCH):
                    go[ci, pl.ds(o, _L)] = jnp.where(
                        pred, cfills[ci], gi[ci, pl.ds(o, _L)])
                return carry

            lax.fori_loop(0, _TROW // _L, jbody, 0, unroll=2)

        m_desc(0).start()
        gin_desc(0, 0).start()
        gin_desc(1, 1).start()
        for t in range(_NT):
            s = t % 2
            u, v = divmod(t, _NV)
            if v == 0:
                m_desc(u).wait()
                if u + 1 < _PWT:
                    m_desc(u + 1).start()
            gin_desc(t, s).wait()
            if t >= 2:
                gout_desc(t - 2, s).wait()
            compute(t, s)
            gout_desc(t, s).start()
            if t + 2 < _NT:
                gin_desc(t + 2, s).start()
        gout_desc(_NT - 2, (_NT - 2) % 2).wait()
        gout_desc(_NT - 1, (_NT - 1) % 2).wait()

    return sc_fill


_sc_fill = _build_sc_call()


def kernel(grid, mask, color):
    gT = jnp.transpose(grid, (1, 2, 3, 0)).reshape(_C, _HW, _B)
    mT = jnp.transpose(mask, (1, 2, 3, 0)).reshape(_HW, _B)
    g5 = gT.reshape(_C, _PT, 8, _BT, 128).transpose(0, 1, 3, 2, 4)
    m4 = mT.reshape(_PT, 8, _BT, 128).transpose(0, 2, 1, 3)

    color = jnp.asarray(color)
    valid = (color >= 0) & (color < _C)
    safe = jnp.clip(color, 0, _C - 1)
    fill = (jnp.arange(_C) == safe).astype(jnp.float32)
    thr = jnp.where(valid, jnp.float32(0.5), jnp.float32(3.0e38))
    params = jnp.concatenate([fill, thr[None]])
    fill16 = jnp.broadcast_to(fill[:, None], (_C, _L)).reshape(_C * _L)
    thr16 = jnp.broadcast_to(thr, (_L,))

    sc5 = _sc_fill(g5, m4, fill16, thr16)
    sc3 = sc5.transpose(0, 1, 3, 2, 4).reshape(_C, _PSC, _B)
    tc_full = _tc_fill(gT, mT, params)
    out3 = lax.dynamic_update_slice(tc_full, sc3, (0, 0, 0))
    return jnp.transpose(out3.reshape(_C, 64, 64, _B), (3, 0, 1, 2))


# speedup vs baseline: 1.4374x; 1.2469x over previous
import jax
import jax.numpy as jnp
from jax.experimental import pallas as pl
from jax.experimental.pallas import tpu as pltpu

_B, _C, _HW = 1024, 10, 4096
_P = 128


def _fill_body(g_ref, m_ref, par_ref, o_ref):
    pred = m_ref[...] > par_ref[_C]
    for c in range(_C):
        o_ref[c] = jnp.where(pred, par_ref[c], g_ref[c])


def _tc_fill(gT, mT, params):
    return pl.pallas_call(
        _fill_body,
        grid=(_HW // _P,),
        in_specs=[
            pl.BlockSpec((_C, _P, _B), lambda j: (0, j, 0)),
            pl.BlockSpec((_P, _B), lambda j: (j, 0)),
            pl.BlockSpec(memory_space=pltpu.SMEM),
        ],
        out_specs=pl.BlockSpec((_C, _P, _B), lambda j: (0, j, 0)),
        out_shape=jax.ShapeDtypeStruct((_C, _HW, _B), jnp.float32),
        compiler_params=pltpu.CompilerParams(
            dimension_semantics=("arbitrary",)),
    )(gT, mT, params)


def kernel(grid, mask, color):
    gT = jnp.transpose(grid, (1, 2, 3, 0)).reshape(_C, _HW, _B)
    mT = jnp.transpose(mask, (1, 2, 3, 0)).reshape(_HW, _B)
    color = jnp.asarray(color)
    valid = (color >= 0) & (color < _C)
    safe = jnp.clip(color, 0, _C - 1)
    fill = (jnp.arange(_C) == safe).astype(jnp.float32)
    thr = jnp.where(valid, jnp.float32(0.5), jnp.float32(3.0e38))
    params = jnp.concatenate([fill, thr[None]])
    out = _tc_fill(gT, mT, params)
    return jnp.transpose(out.reshape(_C, 64, 64, _B), (3, 0, 1, 2))
